# baseline (speedup 1.0000x reference)
import functools

import jax
import jax.numpy as jnp
from jax import lax
from jax.experimental import pallas as pl
from jax.experimental.pallas import tpu as pltpu
from jax.experimental.pallas import tpu_sc as plsc

D_VOCAB = 1000000
D_EMB = 64
B_TOK = 1024 * 200

NC, NS = 2, 16
NW = NC * NS

TCH = 512
N_FULL = D_VOCAB // TCH
TAIL = D_VOCAB - N_FULL * TCH
N_CHUNK = N_FULL + 1
K_MAX = (N_CHUNK + NW - 1) // NW

ROWS_PER_W = B_TOK // NW
IDX_MINOR = 128
CHUNKS_PER_W = ROWS_PER_W // IDX_MINOR

_SC_PARAMS = pltpu.CompilerParams(
    use_tc_tiling_on_sc=False, needs_layout_passes=False
)


def _shuffle(tin, tout, ncols):
    lanes = lax.iota(jnp.int32, 16)

    def vrow(v):
        vv = jnp.full((16,), v, jnp.int32)
        for g in range(4):
            vals = plsc.load_gather(tin, [lanes + (16 * g), vv])
            tout[v, pl.ds(16 * g, 16)] = vals

    def body(vg, _):
        v0 = vg * 4
        vrow(v0)
        vrow(v0 + 1)
        vrow(v0 + 2)
        vrow(v0 + 3)
        return _

    lax.fori_loop(0, ncols // 4, body, None)


def _make_transpose():
    mesh = plsc.VectorSubcoreMesh(core_axis_name="c", subcore_axis_name="s")

    @functools.partial(
        pl.kernel,
        mesh=mesh,
        out_type=jax.ShapeDtypeStruct((D_VOCAB, D_EMB), jnp.float32),
        compiler_params=_SC_PARAMS,
        scratch_types=[
            pltpu.VMEM((D_EMB, TCH), jnp.float32),
            pltpu.VMEM((TCH, D_EMB), jnp.float32),
            pltpu.SemaphoreType.DMA,
        ],
    )
    def transpose(w_hbm, table_hbm, tin, tout, sem):
        wid = lax.axis_index("s") * NC + lax.axis_index("c")

        def stage(base, ncols):
            def fire(d, _):
                pltpu.async_copy(
                    w_hbm.at[d, pl.ds(base, ncols)], tin.at[d, pl.ds(0, ncols)], sem
                )
                return _

            lax.fori_loop(0, D_EMB, fire, None)
            pltpu.make_async_copy(
                w_hbm.at[pl.ds(0, D_EMB), pl.ds(0, ncols)],
                tin.at[:, pl.ds(0, ncols)],
                sem,
            ).wait()

        def step(k, _):
            c = wid + k * NW

            @pl.when(c < N_FULL)
            def _full():
                base = c * TCH
                stage(base, TCH)
                _shuffle(tin, tout, TCH)
                pltpu.sync_copy(tout, table_hbm.at[pl.ds(base, TCH)])

            @pl.when(c == N_FULL)
            def _tail():
                base = N_FULL * TCH
                stage(base, TAIL)
                _shuffle(tin, tout, TAIL)
                pltpu.sync_copy(
                    tout.at[pl.ds(0, TAIL)], table_hbm.at[pl.ds(base, TAIL)]
                )

            return _

        lax.fori_loop(0, K_MAX, step, None)

    return transpose


def _make_gather():
    mesh = plsc.VectorSubcoreMesh(core_axis_name="c", subcore_axis_name="s")

    @functools.partial(
        pl.kernel,
        mesh=mesh,
        out_type=jax.ShapeDtypeStruct((B_TOK, D_EMB), jnp.float32),
        compiler_params=_SC_PARAMS,
        scratch_types=[
            pltpu.VMEM((CHUNKS_PER_W, IDX_MINOR), jnp.int32),
            pltpu.VMEM((IDX_MINOR, D_EMB), jnp.float32),
            pltpu.SemaphoreType.DMA,
        ],
    )
    def gather(table_hbm, idx_hbm, out_hbm, idx_v, rows_v, sem):
        wid = lax.axis_index("s") * NC + lax.axis_index("c")
        base_chunk = wid * CHUNKS_PER_W
        pltpu.sync_copy(idx_hbm.at[wid], idx_v)

        def body(c, _):
            pltpu.async_copy(table_hbm.at[idx_v.at[c]], rows_v, sem).wait()
            pltpu.sync_copy(
                rows_v, out_hbm.at[pl.ds((base_chunk + c) * IDX_MINOR, IDX_MINOR)]
            )
            return _

        lax.fori_loop(0, CHUNKS_PER_W, body, None)

    return gather


_transpose = _make_transpose()
_gather = _make_gather()


def kernel(x, W_E):
    table = _transpose(W_E)
    idx = x.reshape(NW, CHUNKS_PER_W, IDX_MINOR).astype(jnp.int32)
    out = _gather(table, idx)
    return out.reshape(1024, 200, D_EMB)


# speedup vs baseline: 14.3036x; 14.3036x over previous
import functools

import jax
import jax.numpy as jnp
from jax import lax
from jax.experimental import pallas as pl
from jax.experimental.pallas import tpu as pltpu
from jax.experimental.pallas import tpu_sc as plsc

D_VOCAB = 1000000
D_EMB = 64
B_TOK = 1024 * 200

_TC_CHUNK = 4096

NC, NS = 2, 16
NW = NC * NS
ROWS_PER_W = B_TOK // NW
IDX_MINOR = 128
CHUNKS_PER_W = ROWS_PER_W // IDX_MINOR

_SC_PARAMS = pltpu.CompilerParams(use_tc_tiling_on_sc=False)


PAIR = 524288


def _transpose_body(wa_ref, wb_ref, o_ref):
    o_ref[:, 0:D_EMB] = wa_ref[...].T
    o_ref[:, D_EMB : 2 * D_EMB] = wb_ref[...].T


def _transpose_table(W_E):
    return pl.pallas_call(
        _transpose_body,
        grid=(PAIR // _TC_CHUNK,),
        in_specs=[
            pl.BlockSpec((D_EMB, _TC_CHUNK), lambda i: (0, i)),
            pl.BlockSpec(
                (D_EMB, _TC_CHUNK),
                lambda i: (0, jnp.minimum(i + PAIR // _TC_CHUNK,
                                          pl.cdiv(D_VOCAB, _TC_CHUNK) - 1)),
            ),
        ],
        out_specs=pl.BlockSpec((_TC_CHUNK, 2 * D_EMB), lambda i: (i, 0)),
        out_shape=jax.ShapeDtypeStruct((PAIR, 2 * D_EMB), jnp.float32),
    )(W_E, W_E)


def _make_gather():
    mesh = plsc.VectorSubcoreMesh(core_axis_name="c", subcore_axis_name="s")

    @functools.partial(
        pl.kernel,
        mesh=mesh,
        out_type=jax.ShapeDtypeStruct((B_TOK, D_EMB), jnp.float32),
        compiler_params=_SC_PARAMS,
        scratch_types=[
            pltpu.VMEM((CHUNKS_PER_W, IDX_MINOR), jnp.int32),
            pltpu.VMEM((IDX_MINOR, D_EMB), jnp.float32),
            pltpu.SemaphoreType.DMA,
        ],
    )
    def gather(table_hbm, idx_hbm, out_hbm, idx_v, rows_v, sem):
        wid = lax.axis_index("s") * NC + lax.axis_index("c")
        base_chunk = wid * CHUNKS_PER_W
        pltpu.sync_copy(idx_hbm.at[wid], idx_v)

        def body(c, _):
            pltpu.async_copy(table_hbm.at[idx_v.at[c]], rows_v, sem).wait()
            pltpu.sync_copy(
                rows_v, out_hbm.at[pl.ds((base_chunk + c) * IDX_MINOR, IDX_MINOR)]
            )
            return _

        lax.fori_loop(0, CHUNKS_PER_W, body, None)

    return gather


_gather = _make_gather()


def kernel(x, W_E):
    table = _transpose_table(W_E).reshape(2 * PAIR, D_EMB)
    x32 = x.reshape(NW, CHUNKS_PER_W, IDX_MINOR).astype(jnp.int32)
    idx = jnp.where(x32 < PAIR, 2 * x32, 2 * x32 - 2 * PAIR + 1)
    out = _gather(table, idx)
    return out.reshape(1024, 200, D_EMB)


# speedup vs baseline: 14.7537x; 1.0315x over previous
import functools

import jax
import jax.numpy as jnp
from jax import lax
from jax.experimental import pallas as pl
from jax.experimental.pallas import tpu as pltpu
from jax.experimental.pallas import tpu_sc as plsc

D_VOCAB = 1000000
D_EMB = 64
B_TOK = 1024 * 200

_TC_CHUNK = 4096

NC, NS = 2, 16
NW = NC * NS
ROWS_PER_W = B_TOK // NW
IDX_MINOR = 128
CHUNKS_PER_W = ROWS_PER_W // IDX_MINOR

_SC_PARAMS = pltpu.CompilerParams(use_tc_tiling_on_sc=False)


PAIR = 524288


def _transpose_body(wa_ref, wb_ref, o_ref):
    o_ref[:, 0:D_EMB] = wa_ref[...].T
    o_ref[:, D_EMB : 2 * D_EMB] = wb_ref[...].T


def _transpose_table(W_E):
    return pl.pallas_call(
        _transpose_body,
        grid=(PAIR // _TC_CHUNK,),
        in_specs=[
            pl.BlockSpec((D_EMB, _TC_CHUNK), lambda i: (0, i)),
            pl.BlockSpec(
                (D_EMB, _TC_CHUNK),
                lambda i: (0, jnp.minimum(i + PAIR // _TC_CHUNK,
                                          pl.cdiv(D_VOCAB, _TC_CHUNK) - 1)),
            ),
        ],
        out_specs=pl.BlockSpec((_TC_CHUNK, 2 * D_EMB), lambda i: (i, 0)),
        out_shape=jax.ShapeDtypeStruct((PAIR, 2 * D_EMB), jnp.float32),
    )(W_E, W_E)


B_ROWS_PER_W = 1024 // NW
P_LEN = 200
P_HALF = P_LEN // 2


def _make_gather():
    mesh = plsc.VectorSubcoreMesh(core_axis_name="c", subcore_axis_name="s")

    @functools.partial(
        pl.kernel,
        mesh=mesh,
        out_type=jax.ShapeDtypeStruct((1024, P_LEN, D_EMB), jnp.float32),
        compiler_params=_SC_PARAMS,
        scratch_types=[
            pltpu.VMEM((2 * B_ROWS_PER_W, P_HALF), jnp.int32),
            pltpu.VMEM((P_LEN, D_EMB), jnp.float32),
            pltpu.SemaphoreType.DMA,
        ],
    )
    def gather(table_hbm, idx_hbm, out_hbm, idx_v, rows_v, sem):
        wid = lax.axis_index("s") * NC + lax.axis_index("c")
        pltpu.sync_copy(idx_hbm.at[wid], idx_v)

        def body(k, _):
            cp0 = pltpu.async_copy(
                table_hbm.at[idx_v.at[2 * k]], rows_v.at[pl.ds(0, P_HALF)], sem
            )
            cp1 = pltpu.async_copy(
                table_hbm.at[idx_v.at[2 * k + 1]],
                rows_v.at[pl.ds(P_HALF, P_HALF)],
                sem,
            )
            cp0.wait()
            cp1.wait()
            pltpu.sync_copy(rows_v, out_hbm.at[wid * B_ROWS_PER_W + k])
            return _

        lax.fori_loop(0, B_ROWS_PER_W, body, None)

    return gather


_gather = _make_gather()


def kernel(x, W_E):
    table = _transpose_table(W_E).reshape(2 * PAIR, D_EMB)
    x32 = x.reshape(NW, 2 * B_ROWS_PER_W, P_HALF).astype(jnp.int32)
    idx = jnp.where(x32 < PAIR, 2 * x32, 2 * x32 - 2 * PAIR + 1)
    return _gather(table, idx)


# speedup vs baseline: 15.8985x; 1.0776x over previous
import functools

import jax
import jax.numpy as jnp
from jax import lax
from jax.experimental import pallas as pl
from jax.experimental.pallas import tpu as pltpu
from jax.experimental.pallas import tpu_sc as plsc

D_VOCAB = 1000000
D_EMB = 64
B_TOK = 1024 * 200

_TC_CHUNK = 8192

NC, NS = 2, 16
NW = NC * NS
ROWS_PER_W = B_TOK // NW
IDX_MINOR = 128
CHUNKS_PER_W = ROWS_PER_W // IDX_MINOR

_SC_PARAMS = pltpu.CompilerParams(use_tc_tiling_on_sc=False)


PAIR = 524288


def _transpose_body(wa_ref, wb_ref, o_ref):
    o_ref[:, 0:D_EMB] = wa_ref[...].T
    o_ref[:, D_EMB : 2 * D_EMB] = wb_ref[...].T


def _transpose_table(W_E):
    return pl.pallas_call(
        _transpose_body,
        grid=(PAIR // _TC_CHUNK,),
        in_specs=[
            pl.BlockSpec((D_EMB, _TC_CHUNK), lambda i: (0, i)),
            pl.BlockSpec(
                (D_EMB, _TC_CHUNK),
                lambda i: (0, jnp.minimum(i + PAIR // _TC_CHUNK,
                                          pl.cdiv(D_VOCAB, _TC_CHUNK) - 1)),
            ),
        ],
        out_specs=pl.BlockSpec((_TC_CHUNK, 2 * D_EMB), lambda i: (i, 0)),
        out_shape=jax.ShapeDtypeStruct((PAIR, 2 * D_EMB), jnp.float32),
    )(W_E, W_E)


B_ROWS_PER_W = 1024 // NW
P_LEN = 200
P_HALF = P_LEN // 2


def _make_gather():
    mesh = plsc.VectorSubcoreMesh(core_axis_name="c", subcore_axis_name="s")

    @functools.partial(
        pl.kernel,
        mesh=mesh,
        out_type=jax.ShapeDtypeStruct((1024, P_LEN, D_EMB), jnp.float32),
        compiler_params=_SC_PARAMS,
        scratch_types=[
            pltpu.VMEM((2 * B_ROWS_PER_W, P_HALF), jnp.int32),
            pltpu.VMEM((P_LEN, D_EMB), jnp.float32),
            pltpu.SemaphoreType.DMA,
        ],
    )
    def gather(table_hbm, idx_hbm, out_hbm, idx_v, rows_v, sem):
        wid = lax.axis_index("s") * NC + lax.axis_index("c")
        pltpu.sync_copy(idx_hbm.at[wid], idx_v)

        def body(k, _):
            cp0 = pltpu.async_copy(
                table_hbm.at[idx_v.at[2 * k]], rows_v.at[pl.ds(0, P_HALF)], sem
            )
            cp1 = pltpu.async_copy(
                table_hbm.at[idx_v.at[2 * k + 1]],
                rows_v.at[pl.ds(P_HALF, P_HALF)],
                sem,
            )
            cp0.wait()
            cp1.wait()
            pltpu.sync_copy(rows_v, out_hbm.at[wid * B_ROWS_PER_W + k])
            return _

        lax.fori_loop(0, B_ROWS_PER_W, body, None)

    return gather


_gather = _make_gather()


def kernel(x, W_E):
    table = _transpose_table(W_E).reshape(2 * PAIR, D_EMB)
    x32 = x.reshape(NW, 2 * B_ROWS_PER_W, P_HALF).astype(jnp.int32)
    idx = jnp.where(x32 < PAIR, 2 * x32, 2 * x32 - 2 * PAIR + 1)
    return _gather(table, idx)


# speedup vs baseline: 16.4670x; 1.0358x over previous
import functools

import jax
import jax.numpy as jnp
from jax import lax
from jax.experimental import pallas as pl
from jax.experimental.pallas import tpu as pltpu
from jax.experimental.pallas import tpu_sc as plsc

D_VOCAB = 1000000
D_EMB = 64
B_TOK = 1024 * 200

_TC_CHUNK = 16384

NC, NS = 2, 16
NW = NC * NS
ROWS_PER_W = B_TOK // NW
IDX_MINOR = 128
CHUNKS_PER_W = ROWS_PER_W // IDX_MINOR

_SC_PARAMS = pltpu.CompilerParams(use_tc_tiling_on_sc=False)


PAIR = 524288


def _transpose_body(wa_ref, wb_ref, o_ref):
    o_ref[:, 0:D_EMB] = wa_ref[...].T
    o_ref[:, D_EMB : 2 * D_EMB] = wb_ref[...].T


def _transpose_table(W_E):
    return pl.pallas_call(
        _transpose_body,
        grid=(PAIR // _TC_CHUNK,),
        in_specs=[
            pl.BlockSpec((D_EMB, _TC_CHUNK), lambda i: (0, i)),
            pl.BlockSpec(
                (D_EMB, _TC_CHUNK),
                lambda i: (0, jnp.minimum(i + PAIR // _TC_CHUNK,
                                          pl.cdiv(D_VOCAB, _TC_CHUNK) - 1)),
            ),
        ],
        out_specs=pl.BlockSpec((_TC_CHUNK, 2 * D_EMB), lambda i: (i, 0)),
        out_shape=jax.ShapeDtypeStruct((PAIR, 2 * D_EMB), jnp.float32),
    )(W_E, W_E)


B_ROWS_PER_W = 1024 // NW
P_LEN = 200
P_HALF = P_LEN // 2


def _make_gather():
    mesh = plsc.VectorSubcoreMesh(core_axis_name="c", subcore_axis_name="s")

    @functools.partial(
        pl.kernel,
        mesh=mesh,
        out_type=jax.ShapeDtypeStruct((1024, P_LEN, D_EMB), jnp.float32),
        compiler_params=_SC_PARAMS,
        scratch_types=[
            pltpu.VMEM((2 * B_ROWS_PER_W, P_HALF), jnp.int32),
            pltpu.VMEM((P_LEN, D_EMB), jnp.float32),
            pltpu.SemaphoreType.DMA,
        ],
    )
    def gather(table_hbm, idx_hbm, out_hbm, idx_v, rows_v, sem):
        wid = lax.axis_index("s") * NC + lax.axis_index("c")
        pltpu.sync_copy(idx_hbm.at[wid], idx_v)

        def body(k, _):
            cp0 = pltpu.async_copy(
                table_hbm.at[idx_v.at[2 * k]], rows_v.at[pl.ds(0, P_HALF)], sem
            )
            cp1 = pltpu.async_copy(
                table_hbm.at[idx_v.at[2 * k + 1]],
                rows_v.at[pl.ds(P_HALF, P_HALF)],
                sem,
            )
            cp0.wait()
            cp1.wait()
            pltpu.sync_copy(rows_v, out_hbm.at[wid * B_ROWS_PER_W + k])
            return _

        lax.fori_loop(0, B_ROWS_PER_W, body, None)

    return gather


_gather = _make_gather()


def kernel(x, W_E):
    table = _transpose_table(W_E).reshape(2 * PAIR, D_EMB)
    x32 = x.reshape(NW, 2 * B_ROWS_PER_W, P_HALF).astype(jnp.int32)
    idx = jnp.where(x32 < PAIR, 2 * x32, 2 * x32 - 2 * PAIR + 1)
    return _gather(table, idx)
